# baseline (device time: 66601 ns/iter reference)
import jax
import jax.numpy as jnp
from jax import lax
from jax.experimental import pallas as pl
from jax.experimental.pallas import tpu as pltpu

N_DEV = 4
SQ = 1024
SKV = 1024
HQ_LOCAL = 8
DH = 128
D_MODEL = 1024
WINDOW = 128
SCALE = 0.08838834764831843

CH = 128
KWc = 384


def kernel(x, Wq, K_ext, V_ext, Wo):
    def body(x_ref, wq_ref, k_ref, v_ref, wo_ref, out_ref,
             q_scr, ctx_scr, kb_scr, vb_scr, abuf, rbufP, rbufM,
             sendP, recvP, sendM, recvM,
             sendPa, recvPa, sendMa, recvMa):
        my_pos = lax.axis_index("i")
        left = lax.rem(my_pos + N_DEV - 1, N_DEV)
        right = lax.rem(my_pos + 1, N_DEV)

        barrier_sem = pltpu.get_barrier_semaphore()
        for nbr in (left, right):
            pl.semaphore_signal(
                barrier_sem, inc=1,
                device_id=(nbr,), device_id_type=pl.DeviceIdType.MESH,
            )
        pl.semaphore_wait(barrier_sem, 2)

        wq_slice = (wq_ref[:, pl.ds(my_pos * D_MODEL, D_MODEL)]
                    * SCALE).astype(jnp.bfloat16)
        q_scr[...] = jnp.dot(x_ref[0].astype(jnp.bfloat16), wq_slice,
                             preferred_element_type=jnp.float32
                             ).astype(jnp.bfloat16)
        wo_slice = wo_ref[pl.ds(my_pos * D_MODEL, D_MODEL), :].astype(
            jnp.bfloat16)
        for h in range(HQ_LOCAL):
            kb_scr[h] = k_ref[0, :, h, :].astype(jnp.bfloat16)
            vb_scr[h] = v_ref[0, :, h, :].astype(jnp.bfloat16)

        def rowsP(c):
            return lax.rem(c + 8, N_DEV) * CH

        def rowsM(c):
            return N_DEV * CH + lax.rem(c + 8, N_DEV) * CH

        def compute_chunk(r0):
            ks = pl.multiple_of(jnp.clip(r0 - WINDOW, 0, SKV - KWc), 128)
            qi = lax.broadcasted_iota(jnp.int32, (CH, KWc), 0) + r0
            ki = lax.broadcasted_iota(jnp.int32, (CH, KWc), 1) + ks
            mask = jnp.abs(qi - ki) <= WINDOW
            for h in range(HQ_LOCAL):
                qh = q_scr[pl.ds(r0, CH), h * DH:(h + 1) * DH]
                kh = kb_scr[h, pl.ds(ks, KWc), :]
                vh = vb_scr[h, pl.ds(ks, KWc), :]
                scores = lax.dot_general(
                    qh, kh, (((1,), (1,)), ((), ())),
                    preferred_element_type=jnp.float32,
                )
                wf = jnp.where(mask, jnp.exp(scores), 0.0)
                denom = jnp.sum(wf, axis=-1, keepdims=True)
                w = wf.astype(jnp.bfloat16)
                ctx_scr[pl.ds(r0, CH), h * DH:(h + 1) * DH] = (jnp.dot(
                    w, vh, preferred_element_type=jnp.float32) / denom
                ).astype(jnp.bfloat16)
            pc = jnp.dot(ctx_scr[pl.ds(r0, CH), :],
                         wo_slice, preferred_element_type=jnp.float32)
            abuf[pl.ds(r0, CH), :] = pc.astype(jnp.bfloat16)

        compute_chunk(rowsP(my_pos))
        compute_chunk(rowsM(my_pos))

        rs_pend = []
        for s in range(N_DEV - 1):
            slot = s % 2
            rp = pltpu.make_async_remote_copy(
                src_ref=abuf.at[pl.ds(rowsP(my_pos - s), CH), :],
                dst_ref=rbufP.at[slot],
                send_sem=sendP.at[slot], recv_sem=recvP.at[slot],
                device_id=(right,), device_id_type=pl.DeviceIdType.MESH,
            )
            rm = pltpu.make_async_remote_copy(
                src_ref=abuf.at[pl.ds(rowsM(my_pos + s), CH), :],
                dst_ref=rbufM.at[slot],
                send_sem=sendM.at[slot], recv_sem=recvM.at[slot],
                device_id=(left,), device_id_type=pl.DeviceIdType.MESH,
            )
            if s >= 2:
                rs_pend[s - 2][0].wait_send()
                rs_pend[s - 2][1].wait_send()
            rp.start()
            rm.start()
            rs_pend.append((rp, rm))
            compute_chunk(rowsP(my_pos - s - 1))
            compute_chunk(rowsM(my_pos + s + 1))
            rp.wait_recv()
            rm.wait_recv()
            abuf[pl.ds(rowsP(my_pos - s - 1), CH), :] += rbufP[slot]
            abuf[pl.ds(rowsM(my_pos + s + 1), CH), :] += rbufM[slot]
        for dp, dm in rs_pend[1:]:
            dp.wait_send()
            dm.wait_send()

        rp_own = rowsP(my_pos + 1)
        rm_own = rowsM(my_pos - 1)
        out_ref[0, pl.ds(rp_own, CH), :] = (
            abuf[pl.ds(rp_own, CH), :].astype(jnp.float32))
        out_ref[0, pl.ds(rm_own, CH), :] = (
            abuf[pl.ds(rm_own, CH), :].astype(jnp.float32))

        ag_pend = []
        for s in range(N_DEV - 1):
            slot = s % 2
            rp = pltpu.make_async_remote_copy(
                src_ref=abuf.at[pl.ds(rowsP(my_pos + 1 - s), CH), :],
                dst_ref=rbufP.at[slot],
                send_sem=sendPa.at[slot], recv_sem=recvPa.at[slot],
                device_id=(right,), device_id_type=pl.DeviceIdType.MESH,
            )
            rm = pltpu.make_async_remote_copy(
                src_ref=abuf.at[pl.ds(rowsM(my_pos - 1 + s), CH), :],
                dst_ref=rbufM.at[slot],
                send_sem=sendMa.at[slot], recv_sem=recvMa.at[slot],
                device_id=(left,), device_id_type=pl.DeviceIdType.MESH,
            )
            if s >= 2:
                ag_pend[s - 2][0].wait_send()
                ag_pend[s - 2][1].wait_send()
            rp.start()
            rm.start()
            ag_pend.append((rp, rm))
            rp.wait_recv()
            rm.wait_recv()
            rcp = rowsP(my_pos - s)
            rcm = rowsM(my_pos + s)
            if s < N_DEV - 2:
                abuf[pl.ds(rcp, CH), :] = rbufP[slot]
                abuf[pl.ds(rcm, CH), :] = rbufM[slot]
            out_ref[0, pl.ds(rcp, CH), :] = rbufP[slot].astype(jnp.float32)
            out_ref[0, pl.ds(rcm, CH), :] = rbufM[slot].astype(jnp.float32)
        for dp, dm in ag_pend[1:]:
            dp.wait_send()
            dm.wait_send()

    return pl.pallas_call(
        body,
        out_shape=jax.ShapeDtypeStruct((1, SQ, D_MODEL), jnp.float32),
        in_specs=[pl.BlockSpec(memory_space=pltpu.VMEM)] * 5,
        out_specs=pl.BlockSpec(memory_space=pltpu.VMEM),
        scratch_shapes=[
            pltpu.VMEM((SQ, D_MODEL), jnp.bfloat16),
            pltpu.VMEM((SQ, D_MODEL), jnp.bfloat16),
            pltpu.VMEM((HQ_LOCAL, SKV, DH), jnp.bfloat16),
            pltpu.VMEM((HQ_LOCAL, SKV, DH), jnp.bfloat16),
            pltpu.VMEM((SQ, D_MODEL), jnp.bfloat16),
            pltpu.VMEM((2, CH, D_MODEL), jnp.bfloat16),
            pltpu.VMEM((2, CH, D_MODEL), jnp.bfloat16),
            pltpu.SemaphoreType.DMA((2,)),
            pltpu.SemaphoreType.DMA((2,)),
            pltpu.SemaphoreType.DMA((2,)),
            pltpu.SemaphoreType.DMA((2,)),
            pltpu.SemaphoreType.DMA((2,)),
            pltpu.SemaphoreType.DMA((2,)),
            pltpu.SemaphoreType.DMA((2,)),
            pltpu.SemaphoreType.DMA((2,)),
        ],
        compiler_params=pltpu.CompilerParams(
            collective_id=0,
            vmem_limit_bytes=100 * 1024 * 1024,
        ),
    )(x, Wq, K_ext, V_ext, Wo)


# device time: 61607 ns/iter; 1.0811x vs baseline; 1.0811x over previous
import jax
import jax.numpy as jnp
from jax import lax
from jax.experimental import pallas as pl
from jax.experimental.pallas import tpu as pltpu

N_DEV = 4
SQ = 1024
SKV = 1024
HQ_LOCAL = 8
DH = 128
D_MODEL = 1024
WINDOW = 128
SCALE = 0.08838834764831843

CH = 128
KWc = 384


def kernel(x, Wq, K_ext, V_ext, Wo):
    def body(x_ref, wq_ref, k_ref, v_ref, wo_ref, out_ref,
             q_scr, ctx_scr, abuf, rbufP, rbufM,
             sendP, recvP, sendM, recvM,
             sendPa, recvPa, sendMa, recvMa):
        my_pos = lax.axis_index("i")
        left = lax.rem(my_pos + N_DEV - 1, N_DEV)
        right = lax.rem(my_pos + 1, N_DEV)

        barrier_sem = pltpu.get_barrier_semaphore()
        for nbr in (left, right):
            pl.semaphore_signal(
                barrier_sem, inc=1,
                device_id=(nbr,), device_id_type=pl.DeviceIdType.MESH,
            )
        pl.semaphore_wait(barrier_sem, 2)

        wq_slice = (wq_ref[:, pl.ds(my_pos * D_MODEL, D_MODEL)]
                    * SCALE).astype(jnp.bfloat16)
        q_scr[...] = jnp.dot(x_ref[0].astype(jnp.bfloat16), wq_slice,
                             preferred_element_type=jnp.float32)
        wo_slice = wo_ref[pl.ds(my_pos * D_MODEL, D_MODEL), :].astype(
            jnp.bfloat16)

        def rowsP(c):
            return lax.rem(c + 8, N_DEV) * CH

        def rowsM(c):
            return N_DEV * CH + lax.rem(c + 8, N_DEV) * CH

        def compute_chunk(r0):
            ks = jnp.clip(r0 - WINDOW, 0, SKV - KWc)
            qi = lax.broadcasted_iota(jnp.int32, (CH, KWc), 0) + r0
            ki = lax.broadcasted_iota(jnp.int32, (CH, KWc), 1) + ks
            mask = jnp.abs(qi - ki) <= WINDOW
            for h in range(HQ_LOCAL):
                qh = q_scr[pl.ds(r0, CH), h * DH:(h + 1) * DH]
                kh = k_ref[0, pl.ds(ks, KWc), h, :]
                vh = v_ref[0, pl.ds(ks, KWc), h, :]
                scores = lax.dot_general(
                    qh, kh, (((1,), (1,)), ((), ())),
                    preferred_element_type=jnp.float32,
                )
                w = jnp.where(mask, jnp.exp(scores), 0.0)
                denom = jnp.sum(w, axis=-1, keepdims=True)
                ctx_scr[pl.ds(r0, CH), h * DH:(h + 1) * DH] = jnp.dot(
                    w, vh, preferred_element_type=jnp.float32) / denom
            pc = jnp.dot(ctx_scr[pl.ds(r0, CH), :].astype(jnp.bfloat16),
                         wo_slice, preferred_element_type=jnp.float32)
            abuf[pl.ds(r0, CH), :] = pc.astype(jnp.bfloat16)

        compute_chunk(rowsP(my_pos))
        compute_chunk(rowsM(my_pos))

        rs_pend = []
        for s in range(N_DEV - 1):
            slot = s % 2
            rp = pltpu.make_async_remote_copy(
                src_ref=abuf.at[pl.ds(rowsP(my_pos - s), CH), :],
                dst_ref=rbufP.at[slot],
                send_sem=sendP.at[slot], recv_sem=recvP.at[slot],
                device_id=(right,), device_id_type=pl.DeviceIdType.MESH,
            )
            rm = pltpu.make_async_remote_copy(
                src_ref=abuf.at[pl.ds(rowsM(my_pos + s), CH), :],
                dst_ref=rbufM.at[slot],
                send_sem=sendM.at[slot], recv_sem=recvM.at[slot],
                device_id=(left,), device_id_type=pl.DeviceIdType.MESH,
            )
            if s >= 2:
                rs_pend[s - 2][0].wait_send()
                rs_pend[s - 2][1].wait_send()
            rp.start()
            rm.start()
            rs_pend.append((rp, rm))
            compute_chunk(rowsP(my_pos - s - 1))
            compute_chunk(rowsM(my_pos + s + 1))
            rp.wait_recv()
            rm.wait_recv()
            abuf[pl.ds(rowsP(my_pos - s - 1), CH), :] += rbufP[slot]
            abuf[pl.ds(rowsM(my_pos + s + 1), CH), :] += rbufM[slot]
        for dp, dm in rs_pend[1:]:
            dp.wait_send()
            dm.wait_send()

        rp_own = rowsP(my_pos + 1)
        rm_own = rowsM(my_pos - 1)
        out_ref[0, pl.ds(rp_own, CH), :] = (
            abuf[pl.ds(rp_own, CH), :].astype(jnp.float32))
        out_ref[0, pl.ds(rm_own, CH), :] = (
            abuf[pl.ds(rm_own, CH), :].astype(jnp.float32))

        ag_pend = []
        for s in range(N_DEV - 1):
            slot = s % 2
            rp = pltpu.make_async_remote_copy(
                src_ref=abuf.at[pl.ds(rowsP(my_pos + 1 - s), CH), :],
                dst_ref=rbufP.at[slot],
                send_sem=sendPa.at[slot], recv_sem=recvPa.at[slot],
                device_id=(right,), device_id_type=pl.DeviceIdType.MESH,
            )
            rm = pltpu.make_async_remote_copy(
                src_ref=abuf.at[pl.ds(rowsM(my_pos - 1 + s), CH), :],
                dst_ref=rbufM.at[slot],
                send_sem=sendMa.at[slot], recv_sem=recvMa.at[slot],
                device_id=(left,), device_id_type=pl.DeviceIdType.MESH,
            )
            if s >= 2:
                ag_pend[s - 2][0].wait_send()
                ag_pend[s - 2][1].wait_send()
            rp.start()
            rm.start()
            ag_pend.append((rp, rm))
            rp.wait_recv()
            rm.wait_recv()
            rcp = rowsP(my_pos - s)
            rcm = rowsM(my_pos + s)
            if s < N_DEV - 2:
                abuf[pl.ds(rcp, CH), :] = rbufP[slot]
                abuf[pl.ds(rcm, CH), :] = rbufM[slot]
            out_ref[0, pl.ds(rcp, CH), :] = rbufP[slot].astype(jnp.float32)
            out_ref[0, pl.ds(rcm, CH), :] = rbufM[slot].astype(jnp.float32)
        for dp, dm in ag_pend[1:]:
            dp.wait_send()
            dm.wait_send()

    return pl.pallas_call(
        body,
        out_shape=jax.ShapeDtypeStruct((1, SQ, D_MODEL), jnp.float32),
        in_specs=[pl.BlockSpec(memory_space=pltpu.VMEM)] * 5,
        out_specs=pl.BlockSpec(memory_space=pltpu.VMEM),
        scratch_shapes=[
            pltpu.VMEM((SQ, D_MODEL), jnp.float32),
            pltpu.VMEM((SQ, D_MODEL), jnp.float32),
            pltpu.VMEM((SQ, D_MODEL), jnp.bfloat16),
            pltpu.VMEM((2, CH, D_MODEL), jnp.bfloat16),
            pltpu.VMEM((2, CH, D_MODEL), jnp.bfloat16),
            pltpu.SemaphoreType.DMA((2,)),
            pltpu.SemaphoreType.DMA((2,)),
            pltpu.SemaphoreType.DMA((2,)),
            pltpu.SemaphoreType.DMA((2,)),
            pltpu.SemaphoreType.DMA((2,)),
            pltpu.SemaphoreType.DMA((2,)),
            pltpu.SemaphoreType.DMA((2,)),
            pltpu.SemaphoreType.DMA((2,)),
        ],
        compiler_params=pltpu.CompilerParams(
            collective_id=0,
            vmem_limit_bytes=100 * 1024 * 1024,
        ),
    )(x, Wq, K_ext, V_ext, Wo)
